# baseline (speedup 1.0000x reference)
import functools

import jax
import jax.numpy as jnp
from jax import lax
from jax.experimental import pallas as pl
from jax.experimental.pallas import tpu as pltpu
from jax.experimental.pallas import tpu_sc as plsc

ENCODING_DIM = 64


def _sc_row_lookup(pos_arr, num_arr, table):
    mesh = plsc.VectorSubcoreMesh(
        core_axis_name="c", subcore_axis_name="s", num_cores=1, num_subcores=1
    )

    @functools.partial(
        pl.kernel,
        out_type=jax.ShapeDtypeStruct((1, ENCODING_DIM), jnp.float32),
        mesh=mesh,
        scratch_types=[
            pltpu.VMEM((16,), jnp.int32),
            pltpu.VMEM((16,), jnp.int32),
            pltpu.VMEM((1, ENCODING_DIM), jnp.float32),
        ],
        compiler_params=pltpu.CompilerParams(
            disable_bounds_checks=True,
            disable_semaphore_checks=True,
            skip_device_barrier=True,
            use_tc_tiling_on_sc=True,
        ),
    )
    def k(pos_hbm, num_hbm, table_hbm, out_hbm, pos_v, num_v, row_v):
        wid = lax.axis_index("c") * 16 + lax.axis_index("s")

        @pl.when(wid == 0)
        def _():
            pltpu.sync_copy(pos_hbm, pos_v.at[pl.ds(0, 1)])
            pltpu.sync_copy(num_hbm, num_v.at[pl.ds(0, 1)])
            s = pos_v[...][0] % num_v[...][0]
            pltpu.sync_copy(table_hbm.at[pl.ds(s, 1)], row_v)
            pltpu.sync_copy(row_v, out_hbm)

    return k(pos_arr, num_arr, table)


def kernel(position, num_players, table):
    pos_arr = jnp.reshape(jnp.asarray(position, jnp.int32), (1,))
    num_arr = jnp.reshape(jnp.asarray(num_players, jnp.int32), (1,))
    out = _sc_row_lookup(pos_arr, num_arr, table)
    return out[0]


# speedup vs baseline: 1.3387x; 1.3387x over previous
import jax
import jax.numpy as jnp
from jax.experimental import pallas as pl
from jax.experimental.pallas import tpu as pltpu

ENCODING_DIM = 64


def _body(pos_s, num_s, table_hbm, out_v, sem):
    s = pos_s[0] % num_s[0]
    pltpu.make_async_copy(table_hbm.at[pl.ds(s, 1)], out_v, sem).start()
    pltpu.make_async_copy(table_hbm.at[pl.ds(s, 1)], out_v, sem).wait()


def kernel(position, num_players, table):
    pos_arr = jnp.reshape(jnp.asarray(position, jnp.int32), (1,))
    num_arr = jnp.reshape(jnp.asarray(num_players, jnp.int32), (1,))
    out = pl.pallas_call(
        _body,
        in_specs=[
            pl.BlockSpec(memory_space=pltpu.SMEM),
            pl.BlockSpec(memory_space=pltpu.SMEM),
            pl.BlockSpec(memory_space=pl.ANY),
        ],
        out_specs=pl.BlockSpec(memory_space=pltpu.VMEM),
        out_shape=jax.ShapeDtypeStruct((1, ENCODING_DIM), jnp.float32),
        scratch_shapes=[pltpu.SemaphoreType.DMA],
    )(pos_arr, num_arr, table)
    return out[0]
